# Initial kernel scaffold; baseline (speedup 1.0000x reference)
import jax
import jax.numpy as jnp
from jax.experimental import pallas as pl


def kernel(x_nchw, w1_oihw, b1, gamma, beta, w2_oihw, b2):
    raise NotImplementedError("write your pallas kernel here")


# speedup vs baseline: 6.2264x; 6.2264x over previous
import functools

import jax
import jax.numpy as jnp
from jax.experimental import pallas as pl
from jax.experimental.pallas import tpu as pltpu

EPS = 1e-5
VMEM_LIMIT_BYTES = 48 * 1024 * 1024


def _conv_stats_kernel(x_ref, w1_ref, y_ref, stats_ref, xp_ref, tap_ref,
                       *, w, hw, pad, tile):
  cin = x_ref.shape[0]
  cmid = w1_ref.shape[0]
  xp_ref[:, :pad] = jnp.zeros((cin, pad), jnp.float32)
  xp_ref[:, pad + hw:] = jnp.zeros((cin, pad), jnp.float32)
  xp_ref[:, pad:pad + hw] = x_ref[...]

  lane_w = jax.lax.broadcasted_iota(jnp.int32, (1, tile), 1) % w
  mask_l = (lane_w > 0).astype(jnp.float32)
  mask_r = (lane_w < w - 1).astype(jnp.float32)

  s = jnp.zeros((cmid, 1), jnp.float32)
  s2 = jnp.zeros((cmid, 1), jnp.float32)
  for t in range(hw // tile):
    base = pad + t * tile
    for k in range(9):
      dy, dx = divmod(k, 3)
      off = base + (dy - 1) * w + (dx - 1)
      sl = xp_ref[:, off:off + tile]
      if dx == 0:
        sl = sl * mask_l
      elif dx == 2:
        sl = sl * mask_r
      tap_ref[k * cin:(k + 1) * cin, :] = sl
    acc = jnp.dot(w1_ref[...], tap_ref[...],
                  preferred_element_type=jnp.float32)
    y_ref[:, t * tile:(t + 1) * tile] = acc
    s = s + jnp.sum(acc, axis=1, keepdims=True)
    s2 = s2 + jnp.sum(acc * acc, axis=1, keepdims=True)
  stats_ref[:, 0:1] = s
  stats_ref[:, 1:2] = s2


def _apply_kernel(y_ref, scale_ref, shift_ref, w2_ref, b2_ref, o_ref):
  h = jnp.maximum(y_ref[...] * scale_ref[...] + shift_ref[...], 0.0)
  z = jnp.dot(w2_ref[...], h, preferred_element_type=jnp.float32) + b2_ref[...]
  o_ref[...] = jax.nn.sigmoid(z).astype(o_ref.dtype)


def kernel(x_nchw, w1_oihw, b1, gamma, beta, w2_oihw, b2):
  del b1
  N, Cin, H, W = x_nchw.shape
  Cmid = w1_oihw.shape[0]
  Cout = w2_oihw.shape[0]
  HW = H * W
  PAD = 2 * W
  TILE = 2048
  count = float(N * H * W)

  x3 = x_nchw.reshape(N, Cin, HW)
  w1s = jnp.transpose(w1_oihw, (0, 2, 3, 1)).reshape(Cmid, 9 * Cin)
  w2m = w2_oihw[:, :, 0, 0]

  cparams = pltpu.CompilerParams(dimension_semantics=("parallel",),
                                 vmem_limit_bytes=VMEM_LIMIT_BYTES)

  y, stats = pl.pallas_call(
      functools.partial(_conv_stats_kernel, w=W, hw=HW, pad=PAD, tile=TILE),
      out_shape=(jax.ShapeDtypeStruct((N, Cmid, HW), jnp.float32),
                 jax.ShapeDtypeStruct((N, Cmid, 2), jnp.float32)),
      grid=(N,),
      in_specs=[
          pl.BlockSpec((None, Cin, HW), lambda n: (n, 0, 0)),
          pl.BlockSpec((Cmid, 9 * Cin), lambda n: (0, 0)),
      ],
      out_specs=(pl.BlockSpec((None, Cmid, HW), lambda n: (n, 0, 0)),
                 pl.BlockSpec((None, Cmid, 2), lambda n: (n, 0, 0))),
      scratch_shapes=[pltpu.VMEM((Cin, PAD + HW + PAD), jnp.float32),
                      pltpu.VMEM((9 * Cin, TILE), jnp.float32)],
      compiler_params=cparams,
      cost_estimate=pl.CostEstimate(
          flops=int((2 * 9 * Cin * Cmid + 4 * Cmid) * N * HW),
          transcendentals=0,
          bytes_accessed=int(4 * (N * Cin * HW + N * Cmid * HW))),
  )(x3, w1s)

  s = jnp.sum(stats[:, :, 0], axis=0)
  s2 = jnp.sum(stats[:, :, 1], axis=0)
  mean = s / count
  var = s2 / count - mean * mean
  scale = gamma * jax.lax.rsqrt(var + EPS)
  shift = beta - mean * scale

  T2 = min(4096, HW)
  cparams2 = pltpu.CompilerParams(
      dimension_semantics=("parallel", "parallel"),
      vmem_limit_bytes=VMEM_LIMIT_BYTES)
  out = pl.pallas_call(
      _apply_kernel,
      out_shape=jax.ShapeDtypeStruct((N, Cout, HW), jnp.float32),
      grid=(N, HW // T2),
      in_specs=[
          pl.BlockSpec((None, Cmid, T2), lambda n, t: (n, 0, t)),
          pl.BlockSpec((Cmid, 1), lambda n, t: (0, 0)),
          pl.BlockSpec((Cmid, 1), lambda n, t: (0, 0)),
          pl.BlockSpec((Cout, Cmid), lambda n, t: (0, 0)),
          pl.BlockSpec((Cout, 1), lambda n, t: (0, 0)),
      ],
      out_specs=pl.BlockSpec((None, Cout, T2), lambda n, t: (n, 0, t)),
      compiler_params=cparams2,
      cost_estimate=pl.CostEstimate(
          flops=int((2 * Cmid * Cout + 2 * Cmid + 2 * Cout) * N * HW),
          transcendentals=int(Cout * N * HW),
          bytes_accessed=int(4 * (N * Cmid * HW + N * Cout * HW))),
  )(y, scale.reshape(Cmid, 1), shift.reshape(Cmid, 1), w2m,
    b2.reshape(Cout, 1))

  return out.reshape(N, Cout, H, W)


# speedup vs baseline: 6.4517x; 1.0362x over previous
import functools

import jax
import jax.numpy as jnp
from jax.experimental import pallas as pl
from jax.experimental.pallas import tpu as pltpu

EPS = 1e-5
VMEM_LIMIT_BYTES = 48 * 1024 * 1024


def _conv_stats_kernel(x_ref, w1_ref, y_ref, stats_ref, xp_ref, tap_ref,
                       *, w, hw, pad, tile):
  cin = x_ref.shape[0]
  cmid = w1_ref.shape[0]
  xp_ref[:, :pad] = jnp.zeros((cin, pad), jnp.bfloat16)
  xp_ref[:, pad + hw:] = jnp.zeros((cin, pad), jnp.bfloat16)
  xp_ref[:, pad:pad + hw] = x_ref[...].astype(jnp.bfloat16)

  lane_w = jax.lax.broadcasted_iota(jnp.int32, (1, tile), 1) % w
  mask_l = (lane_w > 0).astype(jnp.bfloat16)
  mask_r = (lane_w < w - 1).astype(jnp.bfloat16)

  s = jnp.zeros((cmid, 1), jnp.float32)
  s2 = jnp.zeros((cmid, 1), jnp.float32)
  for t in range(hw // tile):
    base = pad + t * tile
    for k in range(9):
      dy, dx = divmod(k, 3)
      off = base + (dy - 1) * w + (dx - 1)
      sl = xp_ref[:, off:off + tile]
      if dx == 0:
        sl = sl * mask_l
      elif dx == 2:
        sl = sl * mask_r
      tap_ref[k * cin:(k + 1) * cin, :] = sl
    acc = jnp.dot(w1_ref[...], tap_ref[...],
                  preferred_element_type=jnp.float32)
    y_ref[:, t * tile:(t + 1) * tile] = acc.astype(y_ref.dtype)
    s = s + jnp.sum(acc, axis=1, keepdims=True)
    s2 = s2 + jnp.sum(acc * acc, axis=1, keepdims=True)
  stats_ref[:, 0:1] = s
  stats_ref[:, 1:2] = s2


def _apply_kernel(y_ref, scale_ref, shift_ref, w2_ref, b2_ref, o_ref):
  y = y_ref[...].astype(jnp.float32)
  h = jnp.maximum(y * scale_ref[...] + shift_ref[...], 0.0)
  z = jnp.dot(w2_ref[...], h, preferred_element_type=jnp.float32) + b2_ref[...]
  o_ref[...] = jax.nn.sigmoid(z).astype(o_ref.dtype)


def kernel(x_nchw, w1_oihw, b1, gamma, beta, w2_oihw, b2):
  del b1
  N, Cin, H, W = x_nchw.shape
  Cmid = w1_oihw.shape[0]
  Cout = w2_oihw.shape[0]
  HW = H * W
  PAD = 2 * W
  TILE = 2048
  count = float(N * H * W)

  x3 = x_nchw.reshape(N, Cin, HW)
  w1s = jnp.transpose(w1_oihw, (0, 2, 3, 1)).reshape(Cmid, 9 * Cin)
  w1s = w1s.astype(jnp.bfloat16)
  w2m = w2_oihw[:, :, 0, 0]

  cparams = pltpu.CompilerParams(dimension_semantics=("parallel",),
                                 vmem_limit_bytes=VMEM_LIMIT_BYTES)

  y, stats = pl.pallas_call(
      functools.partial(_conv_stats_kernel, w=W, hw=HW, pad=PAD, tile=TILE),
      out_shape=(jax.ShapeDtypeStruct((N, Cmid, HW), jnp.bfloat16),
                 jax.ShapeDtypeStruct((N, Cmid, 2), jnp.float32)),
      grid=(N,),
      in_specs=[
          pl.BlockSpec((None, Cin, HW), lambda n: (n, 0, 0)),
          pl.BlockSpec((Cmid, 9 * Cin), lambda n: (0, 0)),
      ],
      out_specs=(pl.BlockSpec((None, Cmid, HW), lambda n: (n, 0, 0)),
                 pl.BlockSpec((None, Cmid, 2), lambda n: (n, 0, 0))),
      scratch_shapes=[pltpu.VMEM((Cin, PAD + HW + PAD), jnp.bfloat16),
                      pltpu.VMEM((9 * Cin, TILE), jnp.bfloat16)],
      compiler_params=cparams,
      cost_estimate=pl.CostEstimate(
          flops=int((2 * 9 * Cin * Cmid + 4 * Cmid) * N * HW),
          transcendentals=0,
          bytes_accessed=int(4 * (N * Cin * HW + N * Cmid * HW))),
  )(x3, w1s)

  s = jnp.sum(stats[:, :, 0], axis=0)
  s2 = jnp.sum(stats[:, :, 1], axis=0)
  mean = s / count
  var = s2 / count - mean * mean
  scale = gamma * jax.lax.rsqrt(var + EPS)
  shift = beta - mean * scale

  T2 = min(4096, HW)
  cparams2 = pltpu.CompilerParams(
      dimension_semantics=("parallel", "parallel"),
      vmem_limit_bytes=VMEM_LIMIT_BYTES)
  out = pl.pallas_call(
      _apply_kernel,
      out_shape=jax.ShapeDtypeStruct((N, Cout, HW), jnp.float32),
      grid=(N, HW // T2),
      in_specs=[
          pl.BlockSpec((None, Cmid, T2), lambda n, t: (n, 0, t)),
          pl.BlockSpec((Cmid, 1), lambda n, t: (0, 0)),
          pl.BlockSpec((Cmid, 1), lambda n, t: (0, 0)),
          pl.BlockSpec((Cout, Cmid), lambda n, t: (0, 0)),
          pl.BlockSpec((Cout, 1), lambda n, t: (0, 0)),
      ],
      out_specs=pl.BlockSpec((None, Cout, T2), lambda n, t: (n, 0, t)),
      compiler_params=cparams2,
      cost_estimate=pl.CostEstimate(
          flops=int((2 * Cmid * Cout + 2 * Cmid + 2 * Cout) * N * HW),
          transcendentals=int(Cout * N * HW),
          bytes_accessed=int(4 * (N * Cmid * HW + N * Cout * HW))),
  )(y, scale.reshape(Cmid, 1), shift.reshape(Cmid, 1), w2m,
    b2.reshape(Cout, 1))

  return out.reshape(N, Cout, H, W)


# speedup vs baseline: 10.6191x; 1.6460x over previous
import functools

import jax
import jax.numpy as jnp
from jax.experimental import pallas as pl
from jax.experimental.pallas import tpu as pltpu

EPS = 1e-5
VMEM_LIMIT_BYTES = 48 * 1024 * 1024


def _conv_stats_kernel(x_ref, w1_ref, y_ref, stats_ref, xp_ref, tap_ref,
                       *, w, hw, pad, tile):
  cin = x_ref.shape[0]
  cmid = w1_ref.shape[0]
  xp_ref[:, :pad] = jnp.zeros((cin, pad), jnp.bfloat16)
  xp_ref[:, pad + hw:] = jnp.zeros((cin, pad), jnp.bfloat16)
  xp_ref[:, pad:pad + hw] = x_ref[...].astype(jnp.bfloat16).reshape(cin, hw)

  lane_w = jax.lax.broadcasted_iota(jnp.int32, (1, tile), 1) % w
  mask_l = (lane_w > 0).astype(jnp.bfloat16)
  mask_r = (lane_w < w - 1).astype(jnp.bfloat16)

  s = jnp.zeros((cmid, 1), jnp.float32)
  s2 = jnp.zeros((cmid, 1), jnp.float32)
  for t in range(hw // tile):
    base = pad + t * tile
    for k in range(9):
      dy, dx = divmod(k, 3)
      off = base + (dy - 1) * w + (dx - 1)
      sl = xp_ref[:, off:off + tile]
      if dx == 0:
        sl = sl * mask_l
      elif dx == 2:
        sl = sl * mask_r
      tap_ref[k * cin:(k + 1) * cin, :] = sl
    acc = jnp.dot(w1_ref[...], tap_ref[...],
                  preferred_element_type=jnp.float32)
    y_ref[:, t * tile:(t + 1) * tile] = acc.astype(y_ref.dtype)
    s = s + jnp.sum(acc, axis=1, keepdims=True)
    s2 = s2 + jnp.sum(acc * acc, axis=1, keepdims=True)
  stats_ref[:, 0:1] = s
  stats_ref[:, 1:2] = s2


def _apply_kernel(y_ref, scale_ref, shift_ref, w2_ref, b2_ref, o_ref):
  cout, r, w = o_ref.shape
  y = y_ref[...].astype(jnp.float32)
  h = jnp.maximum(y * scale_ref[...] + shift_ref[...], 0.0)
  z = jnp.dot(w2_ref[...], h, preferred_element_type=jnp.float32) + b2_ref[...]
  o_ref[...] = jax.nn.sigmoid(z).astype(o_ref.dtype).reshape(cout, r, w)


def kernel(x_nchw, w1_oihw, b1, gamma, beta, w2_oihw, b2):
  del b1
  N, Cin, H, W = x_nchw.shape
  Cmid = w1_oihw.shape[0]
  Cout = w2_oihw.shape[0]
  HW = H * W
  PAD = 2 * W
  TILE = 2048
  count = float(N * H * W)

  w1s = jnp.transpose(w1_oihw, (0, 2, 3, 1)).reshape(Cmid, 9 * Cin)
  w1s = w1s.astype(jnp.bfloat16)
  w2m = w2_oihw[:, :, 0, 0]

  cparams = pltpu.CompilerParams(dimension_semantics=("parallel",),
                                 vmem_limit_bytes=VMEM_LIMIT_BYTES)

  y, stats = pl.pallas_call(
      functools.partial(_conv_stats_kernel, w=W, hw=HW, pad=PAD, tile=TILE),
      out_shape=(jax.ShapeDtypeStruct((N, Cmid, HW), jnp.bfloat16),
                 jax.ShapeDtypeStruct((N, Cmid, 2), jnp.float32)),
      grid=(N,),
      in_specs=[
          pl.BlockSpec((None, Cin, H, W), lambda n: (n, 0, 0, 0)),
          pl.BlockSpec((Cmid, 9 * Cin), lambda n: (0, 0)),
      ],
      out_specs=(pl.BlockSpec((None, Cmid, HW), lambda n: (n, 0, 0)),
                 pl.BlockSpec((None, Cmid, 2), lambda n: (n, 0, 0))),
      scratch_shapes=[pltpu.VMEM((Cin, PAD + HW + PAD), jnp.bfloat16),
                      pltpu.VMEM((9 * Cin, TILE), jnp.bfloat16)],
      compiler_params=cparams,
      cost_estimate=pl.CostEstimate(
          flops=int((2 * 9 * Cin * Cmid + 4 * Cmid) * N * HW),
          transcendentals=0,
          bytes_accessed=int(4 * (N * Cin * HW + N * Cmid * HW))),
  )(x_nchw, w1s)

  s = jnp.sum(stats[:, :, 0], axis=0)
  s2 = jnp.sum(stats[:, :, 1], axis=0)
  mean = s / count
  var = s2 / count - mean * mean
  scale = gamma * jax.lax.rsqrt(var + EPS)
  shift = beta - mean * scale

  T2 = min(4096, HW)
  R2 = T2 // W
  cparams2 = pltpu.CompilerParams(
      dimension_semantics=("parallel", "parallel"),
      vmem_limit_bytes=VMEM_LIMIT_BYTES)
  out = pl.pallas_call(
      _apply_kernel,
      out_shape=jax.ShapeDtypeStruct((N, Cout, H, W), jnp.float32),
      grid=(N, HW // T2),
      in_specs=[
          pl.BlockSpec((None, Cmid, T2), lambda n, t: (n, 0, t)),
          pl.BlockSpec((Cmid, 1), lambda n, t: (0, 0)),
          pl.BlockSpec((Cmid, 1), lambda n, t: (0, 0)),
          pl.BlockSpec((Cout, Cmid), lambda n, t: (0, 0)),
          pl.BlockSpec((Cout, 1), lambda n, t: (0, 0)),
      ],
      out_specs=pl.BlockSpec((None, Cout, R2, W), lambda n, t: (n, 0, t, 0)),
      compiler_params=cparams2,
      cost_estimate=pl.CostEstimate(
          flops=int((2 * Cmid * Cout + 2 * Cmid + 2 * Cout) * N * HW),
          transcendentals=int(Cout * N * HW),
          bytes_accessed=int(4 * (N * Cmid * HW + N * Cout * HW))),
  )(y, scale.reshape(Cmid, 1), shift.reshape(Cmid, 1), w2m,
    b2.reshape(Cout, 1))

  return out
